# baseline (speedup 1.0000x reference)
import functools

import jax
import jax.numpy as jnp
from jax import lax
from jax.experimental import pallas as pl
from jax.experimental.pallas import tpu as pltpu
from jax.experimental.pallas import tpu_sc as plsc

B, K, IN_DIM, OUT_DIM = 1024, 128, 1024, 100000
NC, NS, L = 2, 16, 16
NW = NC * NS
S = B // NW
CH = 32
NCH = K // CH
RG = 16
ND = IN_DIM // L
T = S * NCH

_mesh = plsc.VectorSubcoreMesh(core_axis_name="c", subcore_axis_name="s")


@functools.partial(
    pl.kernel,
    mesh=_mesh,
    out_type=jax.ShapeDtypeStruct((B, K), jnp.float32),
    scratch_types=[
        pltpu.VMEM((S, K), jnp.int32),
        pltpu.VMEM((S, IN_DIM), jnp.float32),
        pltpu.VMEM((S, K), jnp.float32),
        pltpu.VMEM((S, K), jnp.float32),
        pltpu.VMEM((2, CH, IN_DIM), jnp.float32),
        pltpu.SemaphoreType.DMA,
        pltpu.SemaphoreType.DMA,
        pltpu.SemaphoreType.DMA,
    ],
)
def _slide_sc(x_hbm, idx_hbm, w_hbm, b_hbm, out_hbm,
              idx_v, x_v, out_v, bias_v, rows_v, sem_b, sem_w0, sem_w1):
    wid = lax.axis_index("s") * NC + lax.axis_index("c")
    base = wid * S

    def w_src(t):
        s = t // NCH
        c = t % NCH
        return w_hbm.at[idx_v.at[s, pl.ds(c * CH, CH)]]

    def start(t, buf, sem):
        pltpu.async_copy(w_src(t), rows_v.at[buf], sem)

    pltpu.sync_copy(idx_hbm.at[pl.ds(base, S)], idx_v)
    start(0, 0, sem_w0)
    start(1, 1, sem_w1)
    pltpu.sync_copy(x_hbm.at[pl.ds(base, S)], x_v)

    def bias_fire(s, carry):
        pltpu.async_copy(b_hbm.at[idx_v.at[s]], bias_v.at[s], sem_b)
        return carry

    lax.fori_loop(0, S, bias_fire, 0)

    lane = lax.iota(jnp.int32, L)
    gd = lax.GatherDimensionNumbers(
        offset_dims=(), collapsed_slice_dims=(0,), start_index_map=(0,))

    def shuf(v, idx):
        return lax.gather(v, idx[:, None], gd, slice_sizes=(1,),
                          mode=lax.GatherScatterMode.PROMISE_IN_BOUNDS)

    def hsum(v):
        for sh in (1, 2, 4, 8):
            v = v + shuf(v, lane ^ sh)
        return v

    def compute(t, buf):
        s = t // NCH
        c = t % NCH
        for r0 in range(0, CH, RG):
            def dot_step(i, accs):
                xv = x_v[s, pl.ds(i * L, L)]
                return tuple(
                    accs[j] + rows_v[buf, r0 + j, pl.ds(i * L, L)] * xv
                    for j in range(RG)
                )
            accs = lax.fori_loop(
                0, ND, dot_step,
                tuple(jnp.zeros((L,), jnp.float32) for _ in range(RG)),
            )
            vec = jnp.zeros((L,), jnp.float32)
            for j in range(RG):
                vec = jnp.where(lane == j, hsum(accs[j]), vec)
            out_v[s, pl.ds(c * CH + r0, L)] = vec

    def pair_body(g, carry):
        t0 = 2 * g
        for buf, sem in ((0, sem_w0), (1, sem_w1)):
            t = t0 + buf
            pltpu.make_async_copy(w_src(t), rows_v.at[buf], sem).wait()
            compute(t, buf)

            @pl.when(t + 2 < T)
            def _():
                start(t + 2, buf, sem)
        return carry

    lax.fori_loop(0, T // 2, pair_body, 0)

    def bias_drain(s, carry):
        pltpu.make_async_copy(b_hbm.at[idx_v.at[s]], bias_v.at[s], sem_b).wait()
        return carry

    lax.fori_loop(0, S, bias_drain, 0)

    def add_bias(t, carry):
        s = t // (K // L)
        j = (t % (K // L)) * L
        out_v[s, pl.ds(j, L)] = out_v[s, pl.ds(j, L)] + bias_v[s, pl.ds(j, L)]
        return carry

    lax.fori_loop(0, S * (K // L), add_bias, 0)

    pltpu.sync_copy(out_v, out_hbm.at[pl.ds(base, S)])


def kernel(in_values, active_out_indices, W, b):
    idx = active_out_indices.astype(jnp.int32)
    return _slide_sc(in_values, idx, W, b)
